# baseline (speedup 1.0000x reference)
import functools

import jax
import jax.numpy as jnp
from jax import lax
from jax.experimental import pallas as pl
from jax.experimental.pallas import tpu as pltpu
from jax.experimental.pallas import tpu_sc as plsc

VOCAB = 1000000
EMB = 32
B = 16384
L = 20
N = B * L

NUM_CORES = 2
NUM_SUBCORES = 16
NW = NUM_CORES * NUM_SUBCORES
B_PER_W = B // NW
ROWS_PER_W = B_PER_W * L
BLOCK_B = 128
N_BLOCKS = B_PER_W // BLOCK_B
SUB_B = 4
SUB = SUB_B * L
N_SUBS = BLOCK_B // SUB_B


def _make_gather():
    mesh = plsc.VectorSubcoreMesh(core_axis_name="c", subcore_axis_name="s")

    @functools.partial(
        pl.kernel,
        mesh=mesh,
        out_type=jax.ShapeDtypeStruct((L, EMB, B), jnp.float32),
        scratch_types=[
            pltpu.VMEM((ROWS_PER_W,), jnp.int32),
            pltpu.VMEM((2, SUB, 128), jnp.float32),
            pltpu.VMEM((L, EMB, BLOCK_B), jnp.float32),
            pltpu.SemaphoreType.DMA((2,)),
            pltpu.SemaphoreType.DMA,
        ],
        compiler_params=pltpu.CompilerParams(needs_layout_passes=False),
    )
    def gather_kernel(idx_hbm, table_hbm, out_hbm, idx_v, rows_v, t_v, gsem, wsem):
        wid = lax.axis_index("s") * NUM_CORES + lax.axis_index("c")
        base = wid * ROWS_PER_W
        b_base = wid * B_PER_W
        pltpu.sync_copy(idx_hbm.at[pl.ds(base, ROWS_PER_W)], idx_v)

        lanes = lax.iota(jnp.int32, 16)

        def gather_args(blk, sub, p):
            return (
                table_hbm.at[
                    idx_v.at[pl.ds(blk * (BLOCK_B * L) + sub * SUB, SUB)]
                ],
                rows_v.at[p],
                gsem.at[p],
            )

        def wb_args(blk, l):
            return (
                t_v.at[l],
                out_hbm.at[l, :, pl.ds(b_base + blk * BLOCK_B, BLOCK_B)],
                wsem,
            )

        @pl.loop(0, N_BLOCKS)
        def _blocks(blk):
            pltpu.async_copy(*gather_args(blk, 0, 0))

            @pl.loop(0, N_SUBS, step=2)
            def _subs(s0):
                for p in range(2):
                    sub = s0 + p
                    pltpu.make_async_copy(*gather_args(blk, sub, p)).wait()

                    @pl.when(sub + 1 < N_SUBS)
                    def _next():
                        pltpu.async_copy(*gather_args(blk, sub + 1, 1 - p))

                    bbase_vec = sub * SUB_B
                    for l in range(L):
                        for b in range(SUB_B):
                            row = b * L + l
                            x0 = rows_v[p, row, pl.ds(0, 16)]
                            x1 = rows_v[p, row, pl.ds(16, 16)]
                            bvec = jnp.full((16,), bbase_vec + b, jnp.int32)
                            plsc.store_scatter(t_v.at[l], [lanes, bvec], x0)
                            plsc.store_scatter(
                                t_v.at[l], [lanes + 16, bvec], x1
                            )

            for l in range(L):
                pltpu.async_copy(*wb_args(blk, l))
            for l in range(L):
                pltpu.make_async_copy(*wb_args(blk, l)).wait()

    return gather_kernel


_gather = _make_gather()


@jax.jit
def kernel(indices, table):
    flat_idx = indices.reshape(N)
    tbl_pad = jnp.pad(table, ((0, 0), (0, 128 - EMB)))
    out_lcb = _gather(flat_idx, tbl_pad)
    return out_lcb.transpose(2, 0, 1)


# speedup vs baseline: 1.0931x; 1.0931x over previous
import functools

import jax
import jax.numpy as jnp
from jax import lax
from jax.experimental import pallas as pl
from jax.experimental.pallas import tpu as pltpu
from jax.experimental.pallas import tpu_sc as plsc

VOCAB = 1000000
EMB = 32
B = 16384
L = 20
N = B * L

NUM_CORES = 2
NUM_SUBCORES = 16
NW = NUM_CORES * NUM_SUBCORES
B_PER_W = B // NW
ROWS_PER_W = B_PER_W * L
CHUNK_B = 16
CHUNK = CHUNK_B * L
N_CHUNKS = B_PER_W // CHUNK_B


def _make_gather():
    mesh = plsc.VectorSubcoreMesh(core_axis_name="c", subcore_axis_name="s")

    @functools.partial(
        pl.kernel,
        mesh=mesh,
        out_type=jax.ShapeDtypeStruct((L, EMB, B), jnp.float32),
        scratch_types=[
            pltpu.VMEM((ROWS_PER_W,), jnp.int32),
            pltpu.VMEM((2, CHUNK, EMB), jnp.float32),
            pltpu.VMEM((2, L, EMB, CHUNK_B), jnp.float32),
            pltpu.SemaphoreType.DMA((2,)),
            pltpu.SemaphoreType.DMA((2,)),
        ],
        compiler_params=pltpu.CompilerParams(
            use_tc_tiling_on_sc=False, needs_layout_passes=False
        ),
    )
    def gather_kernel(idx_hbm, table_hbm, out_hbm, idx_v, rows_v, t_v, gsem, wsem):
        wid = lax.axis_index("s") * NUM_CORES + lax.axis_index("c")
        base = wid * ROWS_PER_W
        b_base = wid * B_PER_W
        pltpu.sync_copy(idx_hbm.at[pl.ds(base, ROWS_PER_W)], idx_v)

        lanes = lax.iota(jnp.int32, 16)

        def gather_args(j, p):
            return (
                table_hbm.at[idx_v.at[pl.ds(j * CHUNK, CHUNK)]],
                rows_v.at[p],
                gsem.at[p],
            )

        pltpu.async_copy(*gather_args(0, 0))

        @pl.loop(0, N_CHUNKS, step=2)
        def _chunks(j0):
            for p in range(2):
                j = j0 + p
                b0 = b_base + j * CHUNK_B
                pltpu.make_async_copy(*gather_args(j, p)).wait()

                @pl.when(j + 1 < N_CHUNKS)
                def _next():
                    pltpu.async_copy(*gather_args(j + 1, (p + 1) % 2))

                @pl.when(j >= 2)
                def _drains():
                    @pl.loop(0, L)
                    def _drain(l):
                        pltpu.make_async_copy(
                            t_v.at[p, l],
                            out_hbm.at[
                                l, :, pl.ds(b_base + (j - 2) * CHUNK_B, CHUNK_B)
                            ],
                            wsem.at[p],
                        ).wait()

                for l in range(L):
                    for bb in range(0, CHUNK_B, 4):
                        xs = []
                        for b in range(bb, bb + 4):
                            row = b * L + l
                            xs.append(rows_v[p, row, pl.ds(0, 16)])
                            xs.append(rows_v[p, row, pl.ds(16, 16)])
                        for k in range(4):
                            b = bb + k
                            bvec = jnp.full((16,), b, jnp.int32)
                            plsc.store_scatter(
                                t_v.at[p, l], [lanes, bvec], xs[2 * k]
                            )
                            plsc.store_scatter(
                                t_v.at[p, l], [lanes + 16, bvec], xs[2 * k + 1]
                            )

                @pl.loop(0, L)
                def _writeback(l):
                    pltpu.async_copy(
                        t_v.at[p, l],
                        out_hbm.at[l, :, pl.ds(b0, CHUNK_B)],
                        wsem.at[p],
                    )

        for j in range(N_CHUNKS - 2, N_CHUNKS):
            p = j % 2

            @pl.loop(0, L)
            def _drain_tail(l):
                pltpu.make_async_copy(
                    t_v.at[p, l],
                    out_hbm.at[l, :, pl.ds(b_base + j * CHUNK_B, CHUNK_B)],
                    wsem.at[p],
                ).wait()

    return gather_kernel


_gather = _make_gather()


@jax.jit
def kernel(indices, table):
    flat_idx = indices.reshape(N)
    out_lcb = _gather(flat_idx, table)
    return out_lcb.transpose(2, 0, 1)
